# Initial kernel scaffold; baseline (speedup 1.0000x reference)
import jax
import jax.numpy as jnp
from jax.experimental import pallas as pl


def kernel(x, segment_ids, W, b):
    raise NotImplementedError("write your pallas kernel here")


# speedup vs baseline: 3.9478x; 3.9478x over previous
import functools
import jax
import jax.numpy as jnp
from jax import lax
from jax.experimental import pallas as pl
from jax.experimental.pallas import tpu as pltpu

B, NF, Fm, H, NC = 8, 4096, 8, 128, 512
NK = 4
CH = NF // NK


def _scores_body(x_ref, w_ref, out_ref):
    xb = x_ref[0]
    xm = jnp.sum(xb, axis=1) * (1.0 / Fm)
    s = lax.dot_general(xm, w_ref[...], (((1,), (0,)), ((), ())),
                        preferred_element_type=jnp.float32)
    out_ref[0, 0, :] = s[:, 0]


def _weights_body(s_ref, seg_ref, w_out_ref):
    s = s_ref[:, 0, :]
    m = jnp.max(s, axis=1, keepdims=True)
    e = jnp.exp(s - m)
    seg = seg_ref[...]
    iota_c = lax.broadcasted_iota(jnp.int32, (NC, 1), 0)
    M = (seg == iota_c).astype(jnp.float32)
    denom = lax.dot_general(e, M, (((1,), (1,)), ((), ())),
                            preferred_element_type=jnp.float32)
    recip = jnp.where(denom > 0, 1.0 / denom, 0.0)
    gath = lax.dot_general(recip, M, (((1,), (0,)), ((), ())),
                           preferred_element_type=jnp.float32)
    w_out_ref[...] = e * gath


def _pool_body(x_ref, w_ref, seg_ref, out_ref):
    k = pl.program_id(1)

    @pl.when(k == 0)
    def _():
        out_ref[...] = jnp.zeros_like(out_ref)

    seg = seg_ref[0]
    wts = w_ref[0]
    iota_c = lax.broadcasted_iota(jnp.int32, (NC, 1), 0)
    Mw = jnp.where(seg == iota_c, wts, 0.0)
    xc = x_ref[0]
    out_ref[0] += lax.dot_general(Mw, xc, (((1,), (0,)), ((), ())),
                                  preferred_element_type=jnp.float32)


def kernel(x, segment_ids, W, b):
    del b
    seg2 = segment_ids.reshape(1, NF).astype(jnp.int32)

    scores = pl.pallas_call(
        _scores_body,
        grid=(B, NK),
        in_specs=[
            pl.BlockSpec((1, CH, Fm, H), lambda bi, ki: (bi, ki, 0, 0)),
            pl.BlockSpec((H, 1), lambda bi, ki: (0, 0)),
        ],
        out_specs=pl.BlockSpec((1, 1, CH), lambda bi, ki: (bi, 0, ki)),
        out_shape=jax.ShapeDtypeStruct((B, 1, NF), jnp.float32),
    )(x, W)

    wts = pl.pallas_call(
        _weights_body,
        in_specs=[
            pl.BlockSpec((B, 1, NF), lambda: (0, 0, 0)),
            pl.BlockSpec((1, NF), lambda: (0, 0)),
        ],
        out_specs=pl.BlockSpec((B, NF), lambda: (0, 0)),
        out_shape=jax.ShapeDtypeStruct((B, NF), jnp.float32),
    )(scores, seg2)

    xr = x.reshape(B, NF, Fm * H)
    seg3 = segment_ids.reshape(NK, 1, CH).astype(jnp.int32)
    pooled = pl.pallas_call(
        _pool_body,
        grid=(B, NK),
        in_specs=[
            pl.BlockSpec((1, CH, Fm * H), lambda bi, ki: (bi, ki, 0)),
            pl.BlockSpec((1, 1, CH), lambda bi, ki: (bi, 0, ki)),
            pl.BlockSpec((1, 1, CH), lambda bi, ki: (ki, 0, 0)),
        ],
        out_specs=pl.BlockSpec((1, NC, Fm * H), lambda bi, ki: (bi, 0, 0)),
        out_shape=jax.ShapeDtypeStruct((B, NC, Fm * H), jnp.float32),
    )(xr, wts.reshape(B, 1, NF), seg3)

    return pooled.reshape(B, NC, Fm, H)
